# Initial kernel scaffold; baseline (speedup 1.0000x reference)
import jax
import jax.numpy as jnp
from jax.experimental import pallas as pl


def kernel(input, target):
    raise NotImplementedError("write your pallas kernel here")


# speedup vs baseline: 8.5022x; 8.5022x over previous
import functools

import jax
import jax.numpy as jnp
from jax import lax
from jax.experimental import pallas as pl
from jax.experimental.pallas import tpu as pltpu
from jax.experimental.pallas import tpu_sc as plsc

LW = 52
LH = 52
NB = 64
NT = 50
NCLS = 80
NCH = 85
CELLS = LW * LH
NTOT = NB * 3 * CELLS
A0W, A0H = 10.0 * LW / 416.0, 13.0 * LH / 416.0
A1W, A1H = 16.0 * LW / 416.0, 30.0 * LH / 416.0
A2W, A2H = 33.0 * LW / 416.0, 23.0 * LH / 416.0
IGNORE = 0.7

NLANE = NCH + 3
FLAT = NB * NT * NLANE
ROWS_PER_TILE = 72
NROWS = ROWS_PER_TILE * 32


def _veltkamp_floor_frac(v, scale):
    c = v * 4097.0
    hi = c - (c - v)
    lo = v - hi
    a = hi * scale
    b = lo * scale
    s = a + b
    n = jnp.floor(s)
    r = (a - n) + b
    n = n + (r >= 1.0).astype(jnp.float32) - (r < 0.0).astype(jnp.float32)
    frac = (a - n) + b
    return n.astype(jnp.int32), frac


def _encode_body(t0, t1, t2, t3, t4, owner_o, fx_o, fy_o, tw_o, th_o,
                 cls_o, s0_o, s1_o, s2_o, idx_o):
    v0, v1, v2, v3, v4 = t0[...], t1[...], t2[...], t3[...], t4[...]
    valid = (v0 + v1 + v2 + v3 + v4) > 0.0
    gi, fx = _veltkamp_floor_frac(v0, float(LW))
    gj, fy = _veltkamp_floor_frac(v1, float(LH))
    gw = v2 * float(LW)
    gh = v3 * float(LH)

    def iou(aw, ah):
        inter = jnp.minimum(gw, aw) * jnp.minimum(gh, ah)
        union = gw * gh + aw * ah - inter + 1e-16
        return inter / union

    i0, i1, i2 = iou(A0W, A0H), iou(A1W, A1H), iou(A2W, A2H)
    best = jnp.where(i0 >= i1, jnp.where(i0 >= i2, 0, 2),
                     jnp.where(i1 >= i2, 1, 2)).astype(jnp.int32)
    supp0 = valid & (i0 > IGNORE)
    supp1 = valid & (i1 > IGNORE)
    supp2 = valid & (i2 > IGNORE)
    aw_b = jnp.where(best == 0, A0W, jnp.where(best == 1, A1W, A2W))
    ah_b = jnp.where(best == 0, A0H, jnp.where(best == 1, A1H, A2H))

    ji = gj * LW + gi
    cell = best * CELLS + ji

    trow = lax.broadcasted_iota(jnp.int32, (NB, NT, NT), 1)
    tcol = lax.broadcasted_iota(jnp.int32, (NB, NT, NT), 2)
    same_cell = cell[:, :, None] == cell[:, None, :]
    valid_col = jnp.broadcast_to(valid[:, None, :], (NB, NT, NT))
    clobbered = jnp.any(same_cell & valid_col & (tcol > trow), axis=2)
    owner = valid & ~clobbered

    same_ji = ji[:, :, None] == ji[:, None, :]
    earlier = tcol < trow

    def first_of(supp_a):
        col = jnp.broadcast_to(supp_a[:, None, :], (NB, NT, NT))
        return supp_a & ~jnp.any(same_ji & earlier & col, axis=2)

    f0, f1, f2 = first_of(supp0), first_of(supp1), first_of(supp2)

    owner_o[...] = owner.astype(jnp.float32)
    fx_o[...] = fx
    fy_o[...] = fy
    tw_o[...] = gw / aw_b
    th_o[...] = gh / ah_b
    cls_o[...] = v4.astype(jnp.int32)
    s0_o[...] = f0.astype(jnp.float32)
    s1_o[...] = f1.astype(jnp.float32)
    s2_o[...] = f2.astype(jnp.float32)

    k = lax.broadcasted_iota(jnp.int32, (NB, NT, NLANE), 2)
    b = lax.broadcasted_iota(jnp.int32, (NB, NT, NLANE), 0)
    best3 = jnp.broadcast_to(best[:, :, None], (NB, NT, NLANE))
    ji3 = jnp.broadcast_to(ji[:, :, None], (NB, NT, NLANE))
    ch = jnp.where(k < NCH, NCH * best3 + k, NCH * (k - NCH) + 4)
    idx_o[...] = (b * 255 + ch) * CELLS + ji3


def _encode(target):
    f2 = jax.ShapeDtypeStruct((NB, NT), jnp.float32)
    i2 = jax.ShapeDtypeStruct((NB, NT), jnp.int32)
    outs = [f2, f2, f2, f2, f2, i2, f2, f2, f2,
            jax.ShapeDtypeStruct((NB, NT, NLANE), jnp.int32)]
    slices = [target[:, :, i] for i in range(5)]
    return pl.pallas_call(_encode_body, out_shape=outs)(*slices)


def _gather_tile(flat_hbm, idx_hbm, out_hbm, idx_v, rows_v, sem):
    wid = lax.axis_index("s") * 2 + lax.axis_index("c")
    r0 = wid * ROWS_PER_TILE
    pltpu.sync_copy(idx_hbm.at[pl.ds(r0, ROWS_PER_TILE)], idx_v)
    def chunk(g, carry):
        copies = []
        for b in range(8):
            j = g * 8 + b
            copies.append(pltpu.make_async_copy(
                flat_hbm.at[idx_v.at[j]], rows_v.at[j], sem))
        for c in copies:
            c.start()
        for c in copies:
            c.wait()
        return carry
    lax.fori_loop(0, ROWS_PER_TILE // 8, chunk, 0)
    pltpu.sync_copy(rows_v, out_hbm.at[pl.ds(r0, ROWS_PER_TILE)])


def _gather(flat_input, idx2d):
    mesh = plsc.VectorSubcoreMesh(core_axis_name="c", subcore_axis_name="s")
    kfn = functools.partial(
        pl.kernel,
        mesh=mesh,
        out_type=jax.ShapeDtypeStruct((NROWS, 128), jnp.float32),
        scratch_types=[
            pltpu.VMEM((ROWS_PER_TILE, 128), jnp.int32),
            pltpu.VMEM((ROWS_PER_TILE, 128), jnp.float32),
            pltpu.SemaphoreType.DMA,
        ],
    )(_gather_tile)
    return kfn(flat_input, idx2d)


def _conf_body(x_ref, o_ref):
    b = pl.program_id(0)
    a = pl.program_id(1)

    @pl.when((b == 0) & (a == 0))
    def _():
        o_ref[0, 0] = 0.0

    z = x_ref[0, 0, :, :]
    p = jax.nn.sigmoid(z)
    term = -jnp.maximum(jnp.log(1.0 - p), -100.0)
    o_ref[0, 0] += jnp.sum(term)


def _conf_sum(input):
    return pl.pallas_call(
        _conf_body,
        grid=(NB, 3),
        in_specs=[pl.BlockSpec((1, 1, LH, LW), lambda b, a: (b, NCH * a + 4, 0, 0))],
        out_specs=pl.BlockSpec((1, 1), lambda b, a: (0, 0),
                               memory_space=pltpu.SMEM),
        out_shape=jax.ShapeDtypeStruct((1, 1), jnp.float32),
    )(input)


def _final_body(g_ref, owner_ref, fx_ref, fy_ref, tw_ref, th_ref, cls_ref,
                s0_ref, s1_ref, s2_ref, sall_ref, o_ref):
    g = g_ref[...]
    of = owner_ref[...]
    k = lax.broadcasted_iota(jnp.int32, (NB, NT, NLANE), 2)

    sig = jax.nn.sigmoid(g)
    logp = jnp.maximum(jnp.log(sig), -100.0)
    log1mp = jnp.maximum(jnp.log(1.0 - sig), -100.0)
    ex = jnp.exp(g)

    def b3(x):
        return jnp.broadcast_to(x[:, :, None], (NB, NT, NLANE))

    of3 = b3(of)
    pred = jnp.where(k < 2, sig, ex)
    tgt = jnp.where(k == 0, b3(fx_ref[...]),
                    jnp.where(k == 1, b3(fy_ref[...]),
                              jnp.where(k == 2, b3(tw_ref[...]), b3(th_ref[...]))))
    mse = of3 * (pred - tgt) * (pred - tgt)
    obj = of3 * (-logp)
    supp = jnp.where(k == NCH, b3(s0_ref[...]),
                     jnp.where(k == NCH + 1, b3(s1_ref[...]), b3(s2_ref[...])))
    noobj_corr = supp * (-log1mp)
    wk = jnp.where(k < 4, 5.0 / NTOT,
                   jnp.where(k == 4, 1.0 / NTOT, -1.0 / NTOT))
    main = jnp.where(k < 4, mse, jnp.where(k == 4, obj, noobj_corr)) * wk
    sum_main = jnp.sum(jnp.where((k < 5) | (k >= NCH), main, 0.0))

    tcls = (k - 5) == b3(cls_ref[...])
    bce = -jnp.where(tcls, logp, log1mp)
    sum_cls = jnp.sum(jnp.where((k >= 5) & (k < NCH), of3 * bce, 0.0))

    npos = jnp.sum(of)
    loss = (sum_main + sall_ref[0, 0] / NTOT + sum_cls / (npos * NCLS)) * NB
    o_ref[0, 0] = loss


def _final(g, owner, fx, fy, tw, th, cls, s0, s1, s2, sall):
    n_in = 10
    return pl.pallas_call(
        _final_body,
        in_specs=[pl.BlockSpec(memory_space=pltpu.VMEM)] * n_in
        + [pl.BlockSpec(memory_space=pltpu.SMEM)],
        out_specs=pl.BlockSpec(memory_space=pltpu.SMEM),
        out_shape=jax.ShapeDtypeStruct((1, 1), jnp.float32),
    )(g, owner, fx, fy, tw, th, cls, s0, s1, s2, sall)


def kernel(input, target):
    owner, fx, fy, tw, th, cls, s0, s1, s2, idx = _encode(target)
    pad = jnp.zeros((NROWS * 128 - FLAT,), jnp.int32)
    idx2d = jnp.concatenate([idx.reshape(-1), pad]).reshape(NROWS, 128)
    g = _gather(input.reshape(-1), idx2d)
    g = g.reshape(-1)[:FLAT].reshape(NB, NT, NLANE)
    sall = _conf_sum(input)
    out = _final(g, owner, fx, fy, tw, th, cls, s0, s1, s2, sall)
    return out[0, 0]
